# baseline (speedup 1.0000x reference)
import jax
import jax.numpy as jnp
from jax.experimental import pallas as pl


def _mask_kernel(mask_ref, out_ref):
    out_ref[...] = (mask_ref[...] != 0.0).astype(jnp.float32)


def kernel(mask):
    rows, cols = mask.shape
    block_rows = 512
    grid = (rows // block_rows,)
    return pl.pallas_call(
        _mask_kernel,
        grid=grid,
        in_specs=[pl.BlockSpec((block_rows, cols), lambda i: (i, 0))],
        out_specs=pl.BlockSpec((block_rows, cols), lambda i: (i, 0)),
        out_shape=jax.ShapeDtypeStruct((rows, cols), jnp.float32),
    )(mask)
